# baseline (speedup 1.0000x reference)
import functools
import numpy as np
import jax
import jax.numpy as jnp
from jax import lax
from jax.experimental import pallas as pl
from jax.experimental.pallas import tpu as pltpu
from jax.experimental.pallas import tpu_sc as plsc

B, N, D = 10, 1000, 128
E, R, H = 320000, 4, 4
DH = D // H
NNODE = B * N
RN = R * NNODE
Q = 4
DQ = D // Q
NB = 10
BLK = NNODE // NB
CHA = 400
KCHA = 25
CHC = 1000
KCHC = 10
NTILES = 32
SH = RN
SLICE = SH // 16
DRAIN = SH // 8


def _attn_body(temb_ref, x_ref, wq_ref, bq_ref, wk_ref, bk_ref, wv_ref, bv_ref,
               wo_ref, bo_ref, o_ref):
    x = x_ref[0]
    q = jnp.dot(x, wq_ref[...], preferred_element_type=jnp.float32) + bq_ref[...]
    k = jnp.dot(x, wk_ref[...], preferred_element_type=jnp.float32) + bk_ref[...]
    v = jnp.dot(x, wv_ref[...], preferred_element_type=jnp.float32) + bv_ref[...]
    outs = []
    for h in range(H):
        qh = q[:, h * DH:(h + 1) * DH]
        kh = k[:, h * DH:(h + 1) * DH]
        vh = v[:, h * DH:(h + 1) * DH]
        s = lax.dot_general(qh, kh, (((1,), (1,)), ((), ())),
                            preferred_element_type=jnp.float32)
        es = jnp.exp(s * (1.0 / np.sqrt(DH)))
        attn = es * (1.0 / jnp.sum(es, axis=-1, keepdims=True))
        outs.append(jnp.dot(attn, vh, preferred_element_type=jnp.float32))
    o = jnp.concatenate(outs, axis=-1)
    o = jnp.dot(o, wo_ref[...], preferred_element_type=jnp.float32) + bo_ref[...]
    o_ref[0] = o + temb_ref[0]


def _time_embedding(t):
    half = D // 2
    freqs = jnp.exp(-jnp.log(10000.0) * jnp.arange(half, dtype=jnp.float32) / half)
    args = t.astype(jnp.float32)[:, None] * freqs[None, :]
    return jnp.concatenate([jnp.sin(args), jnp.cos(args)], axis=-1)


def _attention(x, t, Wq, bq, Wk, bk, Wv, bv, Wo, bo):
    return pl.pallas_call(
        _attn_body,
        grid=(B,),
        in_specs=[
            pl.BlockSpec((1, 1, D), lambda b: (b, 0, 0)),
            pl.BlockSpec((1, N, D), lambda b: (b, 0, 0)),
            pl.BlockSpec((D, D), lambda b: (0, 0)),
            pl.BlockSpec((D,), lambda b: (0,)),
            pl.BlockSpec((D, D), lambda b: (0, 0)),
            pl.BlockSpec((D,), lambda b: (0,)),
            pl.BlockSpec((D, D), lambda b: (0, 0)),
            pl.BlockSpec((D,), lambda b: (0,)),
            pl.BlockSpec((D, D), lambda b: (0, 0)),
            pl.BlockSpec((D,), lambda b: (0,)),
        ],
        out_specs=pl.BlockSpec((1, N, D), lambda b: (b, 0, 0)),
        out_shape=jax.ShapeDtypeStruct((B, N, D), jnp.float32),
    )(_time_embedding(t).reshape(B, 1, D), x, Wq, bq, Wk, bk, Wv, bv, Wo, bo)


def _tables_body(nin, refs):
    h_refs = refs[:nin * 2:2]
    w_refs = refs[1:nin * 2:2]
    out_refs = refs[nin * 2:]
    acc = jnp.zeros((BLK, D), jnp.float32)
    for h_ref, w_ref in zip(h_refs, w_refs):
        acc = acc + jnp.dot(h_ref[...], w_ref[0],
                            preferred_element_type=jnp.float32)
    for q in range(Q):
        out_refs[q][...] = acc[:, q * DQ:(q + 1) * DQ]


def _tables(pairs):
    nin = len(pairs)
    in_specs = []
    args = []
    for h, w in pairs:
        in_specs.append(pl.BlockSpec((BLK, h.shape[1]), lambda r, i: (i, 0)))
        in_specs.append(pl.BlockSpec((1,) + w.shape[1:], lambda r, i: (r, 0, 0)))
        args.extend([h, w])
    def body2(*refs):
        _tables_body(nin, refs)

    return pl.pallas_call(
        body2,
        grid=(R, NB),
        in_specs=in_specs,
        out_specs=[pl.BlockSpec((BLK, DQ), lambda r, i: (r * NB + i, 0))
                   for _ in range(Q)],
        out_shape=[jax.ShapeDtypeStruct((RN, DQ), jnp.float32)
                   for _ in range(Q)],
    )(*args)


def _root_body(nin, refs):
    h_refs = refs[:nin * 2:2]
    w_refs = refs[1:nin * 2:2]
    b_ref = refs[nin * 2]
    out_ref = refs[nin * 2 + 1]
    acc = jnp.zeros((BLK, D), jnp.float32) + b_ref[...]
    for h_ref, w_ref in zip(h_refs, w_refs):
        acc = acc + jnp.dot(h_ref[...], w_ref[...],
                            preferred_element_type=jnp.float32)
    out_ref[...] = acc


def _root(pairs, bias):
    in_specs = []
    args = []
    for h, w in pairs:
        in_specs.append(pl.BlockSpec((BLK, h.shape[1]), lambda i: (i, 0)))
        in_specs.append(pl.BlockSpec(w.shape, lambda i: (0, 0)))
        args.extend([h, w])
    in_specs.append(pl.BlockSpec((D,), lambda i: (0,)))
    args.append(bias)
    nin = len(pairs)

    def body2(*refs):
        _root_body(nin, refs)

    return pl.pallas_call(
        body2,
        grid=(NB,),
        in_specs=in_specs,
        out_specs=pl.BlockSpec((BLK, D), lambda i: (i, 0)),
        out_shape=jax.ShapeDtypeStruct((NNODE, D), jnp.float32),
    )(*args)


@functools.cache
def _sc_mesh():
    return plsc.VectorSubcoreMesh(core_axis_name="c", subcore_axis_name="s",
                                  num_cores=2, num_subcores=16)


def _sc_counts_body(sidx_hbm, zc_hbm, ones_hbm, cnt_out, sidx_v, ones_v,
                    cnt_sh, sem):
    c = lax.axis_index("c")
    s = lax.axis_index("s")
    cb = c * (KCHC * 16) + s * KCHC
    base = s * SLICE
    pltpu.sync_copy(sidx_hbm.at[pl.ds(cb, KCHC)], sidx_v)
    pltpu.sync_copy(ones_hbm, ones_v)
    pltpu.sync_copy(zc_hbm, cnt_sh.at[pl.ds(base, SLICE)])
    plsc.subcore_barrier()
    for k in range(KCHC):
        pltpu.sync_copy(ones_v, cnt_sh.at[sidx_v.at[k]], add=True)
    plsc.subcore_barrier()

    @pl.when(s < 8)
    def _():
        pltpu.sync_copy(cnt_sh.at[pl.ds(s * DRAIN, DRAIN)],
                        cnt_out.at[c, pl.ds(s * DRAIN, DRAIN)])


def _sc_counts(sidx):
    zc = jnp.zeros((SLICE, 8), jnp.float32)
    ones = jnp.ones((CHC, 8), jnp.float32)
    kern = pl.kernel(
        _sc_counts_body,
        out_type=[jax.ShapeDtypeStruct((2, SH, 8), jnp.float32)],
        mesh=_sc_mesh(),
        compiler_params=pltpu.CompilerParams(use_tc_tiling_on_sc=False),
        scratch_types=[
            pltpu.VMEM((KCHC, CHC), jnp.int32),
            pltpu.VMEM((CHC, 8), jnp.float32),
            pltpu.VMEM_SHARED((SH, 8), jnp.float32),
            pltpu.SemaphoreType.DMA,
        ],
    )
    return kern(sidx, zc, ones)[0]


def _sc_agg_body(t0, t1, t2, t3, gidx_hbm, sidx_hbm, za_hbm, acc_out,
                 gidx_v, sidx_v, rows0, rows1, acc_sh, gsem):
    c = lax.axis_index("c")
    s = lax.axis_index("s")
    cb = c * (KCHA * 16) + s * KCHA
    base = s * SLICE
    pltpu.sync_copy(gidx_hbm.at[pl.ds(cb, KCHA)], gidx_v)
    pltpu.sync_copy(sidx_hbm.at[pl.ds(cb, KCHA)], sidx_v)
    pltpu.sync_copy(za_hbm, acc_sh.at[pl.ds(base, SLICE)])
    plsc.subcore_barrier()
    tabs = (t0, t1, t2, t3)
    npairs = (KCHA - 1) // 2
    for q in range(Q):
        tab = tabs[q]

        def gather(k, buf):
            return pltpu.async_copy(tab.at[gidx_v.at[k]], buf, gsem)

        def gwait(buf):
            pltpu.make_async_copy(tab.at[gidx_v.at[0]], buf, gsem).wait()

        gather(0, rows0)
        gather(1, rows1)

        def pair(j, carry):
            a = 2 * j
            gwait(rows0)
            pltpu.sync_copy(rows0, acc_sh.at[sidx_v.at[a]], add=True)
            gather(a + 2, rows0)
            gwait(rows1)
            pltpu.sync_copy(rows1, acc_sh.at[sidx_v.at[a + 1]], add=True)

            @pl.when(j < npairs - 1)
            def _():
                gather(a + 3, rows1)

            return carry

        lax.fori_loop(0, npairs, pair, 0, unroll=False)
        gwait(rows0)
        pltpu.sync_copy(rows0, acc_sh.at[sidx_v.at[KCHA - 1]], add=True)
        plsc.subcore_barrier()

        @pl.when(s < 8)
        def _():
            pltpu.sync_copy(acc_sh.at[pl.ds(s * DRAIN, DRAIN)],
                            acc_out.at[c, q, pl.ds(s * DRAIN, DRAIN)])
        if q < Q - 1:
            plsc.subcore_barrier()
            pltpu.sync_copy(za_hbm, acc_sh.at[pl.ds(base, SLICE)])
        plsc.subcore_barrier()


def _sc_agg(tabs, gidx, sidx):
    za = jnp.zeros((SLICE, DQ), jnp.float32)
    kern = pl.kernel(
        _sc_agg_body,
        out_type=[jax.ShapeDtypeStruct((2, Q, SH, DQ), jnp.float32)],
        mesh=_sc_mesh(),
        compiler_params=pltpu.CompilerParams(use_tc_tiling_on_sc=False),
        scratch_types=[
            pltpu.VMEM((KCHA, CHA), jnp.int32),
            pltpu.VMEM((KCHA, CHA), jnp.int32),
            pltpu.VMEM((CHA, DQ), jnp.float32),
            pltpu.VMEM((CHA, DQ), jnp.float32),
            pltpu.VMEM_SHARED((SH, DQ), jnp.float32),
            pltpu.SemaphoreType.DMA,
        ],
    )
    return kern(tabs[0], tabs[1], tabs[2], tabs[3], gidx, sidx, za)[0]


def _combine_body(acc_ref, cnt_ref, root_ref, out_ref):
    res = root_ref[...]
    cols = []
    for q in range(Q):
        sq = jnp.zeros((BLK, DQ), jnp.float32)
        for r in range(R):
            cr = cnt_ref[0, r, :, 0:1] + cnt_ref[1, r, :, 0:1]
            inv = 1.0 / jnp.maximum(cr, 1.0)
            sq = sq + (acc_ref[0, q, r] + acc_ref[1, q, r]) * inv
        cols.append(sq)
    out_ref[...] = res + jnp.concatenate(cols, axis=-1)


def _combine(acc, cnt, root):
    acc5 = acc.reshape(2, Q, R, NNODE, DQ)
    cnt4 = cnt.reshape(2, R, NNODE, 8)
    return pl.pallas_call(
        _combine_body,
        grid=(NB,),
        in_specs=[
            pl.BlockSpec((2, Q, R, BLK, DQ), lambda i: (0, 0, 0, i, 0)),
            pl.BlockSpec((2, R, BLK, 8), lambda i: (0, 0, i, 0)),
            pl.BlockSpec((BLK, D), lambda i: (i, 0)),
        ],
        out_specs=pl.BlockSpec((BLK, D), lambda i: (i, 0)),
        out_shape=jax.ShapeDtypeStruct((NNODE, D), jnp.float32),
    )(acc5, cnt4, root)


def kernel(x, t, obj_cond, edge_cond, relation_cond, Wq, bq, Wk, bk, Wv, bv,
           Wo, bo, enc_W_rel, enc_W_root, enc_b, fus_W_rel, fus_W_root, fus_b):
    src = edge_cond[0]
    dst = edge_cond[1]
    rel = relation_cond.astype(jnp.int32)
    gidx = (rel * NNODE + src.astype(jnp.int32)).reshape(E // CHA, CHA)
    sidxf = rel * NNODE + dst.astype(jnp.int32)
    sidx = sidxf.reshape(E // CHA, CHA)
    sidxc = sidxf.reshape(E // CHC, CHC)

    cnt = _sc_counts(sidxc)
    tabs1 = _tables([(obj_cond, enc_W_rel)])
    root1 = _root([(obj_cond, enc_W_root)], enc_b)
    acc1 = _sc_agg(tabs1, gidx, sidx)
    xo = _attention(x, t, Wq, bq, Wk, bk, Wv, bv, Wo, bo)
    xo_flat = xo.reshape(NNODE, D)
    g = _combine(acc1, cnt, root1)

    Wa = fus_W_rel[:, :D, :]
    Wb = fus_W_rel[:, D:, :]
    tabs2 = _tables([(xo_flat, Wa), (g, Wb)])
    root2 = _root([(xo_flat, fus_W_root[:D]), (g, fus_W_root[D:])], fus_b)
    acc2 = _sc_agg(tabs2, gidx, sidx)
    out = _combine(acc2, cnt, root2)
    return out.reshape(B, N, D)


# speedup vs baseline: 1.1498x; 1.1498x over previous
import functools
import numpy as np
import jax
import jax.numpy as jnp
from jax import lax
from jax.experimental import pallas as pl
from jax.experimental.pallas import tpu as pltpu
from jax.experimental.pallas import tpu_sc as plsc

B, N, D = 10, 1000, 128
E, R, H = 320000, 4, 4
DH = D // H
NNODE = B * N
RN = R * NNODE
Q = 4
DQ = D // Q
NB = 10
BLK = NNODE // NB
CHA = 400
KCHA = 25
CHC = 1000
KCHC = 10
NTILES = 32
SH = RN
SLICE = SH // 16
DRAIN = SH // 8


def _attn_body(temb_ref, x_ref, wq_ref, bq_ref, wk_ref, bk_ref, wv_ref, bv_ref,
               wo_ref, bo_ref, o_ref):
    x = x_ref[0]
    q = jnp.dot(x, wq_ref[...], preferred_element_type=jnp.float32) + bq_ref[...]
    k = jnp.dot(x, wk_ref[...], preferred_element_type=jnp.float32) + bk_ref[...]
    v = jnp.dot(x, wv_ref[...], preferred_element_type=jnp.float32) + bv_ref[...]
    outs = []
    for h in range(H):
        qh = q[:, h * DH:(h + 1) * DH]
        kh = k[:, h * DH:(h + 1) * DH]
        vh = v[:, h * DH:(h + 1) * DH]
        s = lax.dot_general(qh, kh, (((1,), (1,)), ((), ())),
                            preferred_element_type=jnp.float32)
        es = jnp.exp(s * (1.0 / np.sqrt(DH)))
        attn = es * (1.0 / jnp.sum(es, axis=-1, keepdims=True))
        outs.append(jnp.dot(attn, vh, preferred_element_type=jnp.float32))
    o = jnp.concatenate(outs, axis=-1)
    o = jnp.dot(o, wo_ref[...], preferred_element_type=jnp.float32) + bo_ref[...]
    o_ref[0] = o + temb_ref[0]


def _time_embedding(t):
    half = D // 2
    freqs = jnp.exp(-jnp.log(10000.0) * jnp.arange(half, dtype=jnp.float32) / half)
    args = t.astype(jnp.float32)[:, None] * freqs[None, :]
    return jnp.concatenate([jnp.sin(args), jnp.cos(args)], axis=-1)


def _attention(x, t, Wq, bq, Wk, bk, Wv, bv, Wo, bo):
    return pl.pallas_call(
        _attn_body,
        grid=(B,),
        in_specs=[
            pl.BlockSpec((1, 1, D), lambda b: (b, 0, 0)),
            pl.BlockSpec((1, N, D), lambda b: (b, 0, 0)),
            pl.BlockSpec((D, D), lambda b: (0, 0)),
            pl.BlockSpec((D,), lambda b: (0,)),
            pl.BlockSpec((D, D), lambda b: (0, 0)),
            pl.BlockSpec((D,), lambda b: (0,)),
            pl.BlockSpec((D, D), lambda b: (0, 0)),
            pl.BlockSpec((D,), lambda b: (0,)),
            pl.BlockSpec((D, D), lambda b: (0, 0)),
            pl.BlockSpec((D,), lambda b: (0,)),
        ],
        out_specs=pl.BlockSpec((1, N, D), lambda b: (b, 0, 0)),
        out_shape=jax.ShapeDtypeStruct((B, N, D), jnp.float32),
    )(_time_embedding(t).reshape(B, 1, D), x, Wq, bq, Wk, bk, Wv, bv, Wo, bo)


def _tables_body(nin, refs):
    h_refs = refs[:nin * 2:2]
    w_refs = refs[1:nin * 2:2]
    out_ref = refs[nin * 2]
    acc = jnp.zeros((BLK, D), jnp.float32)
    for h_ref, w_ref in zip(h_refs, w_refs):
        acc = acc + jnp.dot(h_ref[...], w_ref[0],
                            preferred_element_type=jnp.float32)
    out_ref[...] = acc


def _tables(pairs):
    nin = len(pairs)
    in_specs = []
    args = []
    for h, w in pairs:
        in_specs.append(pl.BlockSpec((BLK, h.shape[1]), lambda r, i: (i, 0)))
        in_specs.append(pl.BlockSpec((1,) + w.shape[1:], lambda r, i: (r, 0, 0)))
        args.extend([h, w])
    def body2(*refs):
        _tables_body(nin, refs)

    return pl.pallas_call(
        body2,
        grid=(R, NB),
        in_specs=in_specs,
        out_specs=pl.BlockSpec((BLK, D), lambda r, i: (r * NB + i, 0)),
        out_shape=jax.ShapeDtypeStruct((RN, D), jnp.float32),
    )(*args)


def _root_body(nin, refs):
    h_refs = refs[:nin * 2:2]
    w_refs = refs[1:nin * 2:2]
    b_ref = refs[nin * 2]
    out_ref = refs[nin * 2 + 1]
    acc = jnp.zeros((BLK, D), jnp.float32) + b_ref[...]
    for h_ref, w_ref in zip(h_refs, w_refs):
        acc = acc + jnp.dot(h_ref[...], w_ref[...],
                            preferred_element_type=jnp.float32)
    out_ref[...] = acc


def _root(pairs, bias):
    in_specs = []
    args = []
    for h, w in pairs:
        in_specs.append(pl.BlockSpec((BLK, h.shape[1]), lambda i: (i, 0)))
        in_specs.append(pl.BlockSpec(w.shape, lambda i: (0, 0)))
        args.extend([h, w])
    in_specs.append(pl.BlockSpec((D,), lambda i: (0,)))
    args.append(bias)
    nin = len(pairs)

    def body2(*refs):
        _root_body(nin, refs)

    return pl.pallas_call(
        body2,
        grid=(NB,),
        in_specs=in_specs,
        out_specs=pl.BlockSpec((BLK, D), lambda i: (i, 0)),
        out_shape=jax.ShapeDtypeStruct((NNODE, D), jnp.float32),
    )(*args)


@functools.cache
def _sc_mesh():
    return plsc.VectorSubcoreMesh(core_axis_name="c", subcore_axis_name="s",
                                  num_cores=2, num_subcores=16)


def _sc_counts_body(sidx_hbm, zc_hbm, ones_hbm, cnt_out, sidx_v, ones_v,
                    cnt_sh, sem):
    c = lax.axis_index("c")
    s = lax.axis_index("s")
    cb = c * (KCHC * 16) + s * KCHC
    base = s * SLICE
    pltpu.sync_copy(sidx_hbm.at[pl.ds(cb, KCHC)], sidx_v)
    pltpu.sync_copy(ones_hbm, ones_v)
    pltpu.sync_copy(zc_hbm, cnt_sh.at[pl.ds(base, SLICE)])
    plsc.subcore_barrier()
    for k in range(KCHC):
        pltpu.sync_copy(ones_v, cnt_sh.at[sidx_v.at[k]], add=True)
    plsc.subcore_barrier()

    @pl.when(s < 8)
    def _():
        pltpu.sync_copy(cnt_sh.at[pl.ds(s * DRAIN, DRAIN)],
                        cnt_out.at[c, pl.ds(s * DRAIN, DRAIN)])


def _sc_counts(sidx):
    zc = jnp.zeros((SLICE, 8), jnp.float32)
    ones = jnp.ones((CHC, 8), jnp.float32)
    kern = pl.kernel(
        _sc_counts_body,
        out_type=[jax.ShapeDtypeStruct((2, SH, 8), jnp.float32)],
        mesh=_sc_mesh(),
        compiler_params=pltpu.CompilerParams(use_tc_tiling_on_sc=False),
        scratch_types=[
            pltpu.VMEM((KCHC, CHC), jnp.int32),
            pltpu.VMEM((CHC, 8), jnp.float32),
            pltpu.VMEM_SHARED((SH, 8), jnp.float32),
            pltpu.SemaphoreType.DMA,
        ],
    )
    return kern(sidx, zc, ones)[0]


def _sc_agg_body(tab, gidx_hbm, sidx_hbm, za_hbm, acc_out,
                 gidx_v, sidx_v, rows0, rows1, acc_sh, gsem):
    c = lax.axis_index("c")
    s = lax.axis_index("s")
    cb = c * (KCHA * 16) + s * KCHA
    base = s * SLICE
    pltpu.sync_copy(sidx_hbm.at[pl.ds(cb, KCHA)], sidx_v)
    pltpu.sync_copy(za_hbm, acc_sh.at[pl.ds(base, SLICE)])
    plsc.subcore_barrier()
    npairs = (KCHA - 1) // 2
    for q in range(Q):
        pltpu.sync_copy(gidx_hbm.at[q, pl.ds(cb, KCHA)], gidx_v)

        def gather(k, buf):
            return pltpu.async_copy(tab.at[gidx_v.at[k]], buf, gsem)

        def gwait(buf):
            pltpu.make_async_copy(tab.at[gidx_v.at[0]], buf, gsem).wait()

        gather(0, rows0)
        gather(1, rows1)

        def pair(j, carry):
            a = 2 * j
            gwait(rows0)
            pltpu.sync_copy(rows0, acc_sh.at[sidx_v.at[a]], add=True)
            gather(a + 2, rows0)
            gwait(rows1)
            pltpu.sync_copy(rows1, acc_sh.at[sidx_v.at[a + 1]], add=True)

            @pl.when(j < npairs - 1)
            def _():
                gather(a + 3, rows1)

            return carry

        lax.fori_loop(0, npairs, pair, 0, unroll=False)
        gwait(rows0)
        pltpu.sync_copy(rows0, acc_sh.at[sidx_v.at[KCHA - 1]], add=True)
        plsc.subcore_barrier()

        @pl.when(s < 8)
        def _():
            pltpu.sync_copy(acc_sh.at[pl.ds(s * DRAIN, DRAIN)],
                            acc_out.at[c, q, pl.ds(s * DRAIN, DRAIN)])
        if q < Q - 1:
            plsc.subcore_barrier()
            pltpu.sync_copy(za_hbm, acc_sh.at[pl.ds(base, SLICE)])
        plsc.subcore_barrier()


def _sc_agg(tab, gidx4, sidx):
    za = jnp.zeros((SLICE, DQ), jnp.float32)
    kern = pl.kernel(
        _sc_agg_body,
        out_type=[jax.ShapeDtypeStruct((2, Q, SH, DQ), jnp.float32)],
        mesh=_sc_mesh(),
        compiler_params=pltpu.CompilerParams(use_tc_tiling_on_sc=False),
        scratch_types=[
            pltpu.VMEM((KCHA, CHA), jnp.int32),
            pltpu.VMEM((KCHA, CHA), jnp.int32),
            pltpu.VMEM((CHA, DQ), jnp.float32),
            pltpu.VMEM((CHA, DQ), jnp.float32),
            pltpu.VMEM_SHARED((SH, DQ), jnp.float32),
            pltpu.SemaphoreType.DMA,
        ],
    )
    return kern(tab.reshape(Q * RN, DQ), gidx4, sidx, za)[0]


def _combine_body(acc_ref, cnt_ref, root_ref, out_ref):
    res = root_ref[...]
    cols = []
    for q in range(Q):
        sq = jnp.zeros((BLK, DQ), jnp.float32)
        for r in range(R):
            cr = cnt_ref[0, r, :, 0:1] + cnt_ref[1, r, :, 0:1]
            inv = 1.0 / jnp.maximum(cr, 1.0)
            sq = sq + (acc_ref[0, q, r] + acc_ref[1, q, r]) * inv
        cols.append(sq)
    out_ref[...] = res + jnp.concatenate(cols, axis=-1)


def _combine(acc, cnt, root):
    acc5 = acc.reshape(2, Q, R, NNODE, DQ)
    cnt4 = cnt.reshape(2, R, NNODE, 8)
    return pl.pallas_call(
        _combine_body,
        grid=(NB,),
        in_specs=[
            pl.BlockSpec((2, Q, R, BLK, DQ), lambda i: (0, 0, 0, i, 0)),
            pl.BlockSpec((2, R, BLK, 8), lambda i: (0, 0, i, 0)),
            pl.BlockSpec((BLK, D), lambda i: (i, 0)),
        ],
        out_specs=pl.BlockSpec((BLK, D), lambda i: (i, 0)),
        out_shape=jax.ShapeDtypeStruct((NNODE, D), jnp.float32),
    )(acc5, cnt4, root)


def kernel(x, t, obj_cond, edge_cond, relation_cond, Wq, bq, Wk, bk, Wv, bv,
           Wo, bo, enc_W_rel, enc_W_root, enc_b, fus_W_rel, fus_W_root, fus_b):
    src = edge_cond[0]
    dst = edge_cond[1]
    rel = relation_cond.astype(jnp.int32)
    gidxf = (rel * NNODE + src.astype(jnp.int32)) * Q
    gidx4 = jnp.stack([gidxf + q for q in range(Q)]).reshape(Q, E // CHA, CHA)
    sidxf = rel * NNODE + dst.astype(jnp.int32)
    sidx = sidxf.reshape(E // CHA, CHA)
    sidxc = sidxf.reshape(E // CHC, CHC)

    cnt = _sc_counts(sidxc)
    tabs1 = _tables([(obj_cond, enc_W_rel)])
    root1 = _root([(obj_cond, enc_W_root)], enc_b)
    acc1 = _sc_agg(tabs1, gidx4, sidx)
    xo = _attention(x, t, Wq, bq, Wk, bk, Wv, bv, Wo, bo)
    xo_flat = xo.reshape(NNODE, D)
    g = _combine(acc1, cnt, root1)

    Wa = fus_W_rel[:, :D, :]
    Wb = fus_W_rel[:, D:, :]
    tabs2 = _tables([(xo_flat, Wa), (g, Wb)])
    root2 = _root([(xo_flat, fus_W_root[:D]), (g, fus_W_root[D:])], fus_b)
    acc2 = _sc_agg(tabs2, gidx4, sidx)
    out = _combine(acc2, cnt, root2)
    return out.reshape(B, N, D)
